# baseline (speedup 1.0000x reference)
import functools

import jax
import jax.numpy as jnp
from jax import lax
from jax.experimental import pallas as pl
from jax.experimental.pallas import tpu as pltpu
from jax.experimental.pallas import tpu_sc as plsc

N = 2048
D = 768
DFF = 2048
E = 8
R = 16

BT = 256
NB = 2 * N // BT + E
S = NB * BT

NW = 32
SPT = S // NW
PPT = 2 * N // NW
TPT = N // NW
GB = 64

_SC_MESH = dict(core_axis_name="c", subcore_axis_name="s")


def _logits_kernel(x_ref, wg_ref, lt_ref):
    lt_ref[...] = jax.lax.dot_general(
        wg_ref[...], x_ref[...], (((1,), (1,)), ((), ())),
        preferred_element_type=jnp.float32)


def _router_logits(x, Wg):
    return pl.pallas_call(
        _logits_kernel,
        out_shape=jax.ShapeDtypeStruct((E, N), jnp.float32),
    )(x, Wg)


def _route_body(lt_hbm, x_hbm, xs_hbm, be_hbm, so_hbm, tw_hbm,
                lt_v, eix_v, wts_v, pos_v, be_v, own_rows, rows_buf, sem):
    cid = lax.axis_index("c")
    sid = lax.axis_index("s")
    wid = sid * 2 + cid
    iota16 = lax.iota(jnp.int32, 16)

    pltpu.sync_copy(lt_hbm, lt_v)

    def route_chunk(c, carry):
        base = c * 16
        ls = [lt_v[pl.ds(e * N + base, 16)] for e in range(E)]
        m1 = ls[0]
        i1 = jnp.zeros((16,), jnp.int32)
        for e in range(1, E):
            upd = ls[e] > m1
            m1 = jnp.where(upd, ls[e], m1)
            i1 = jnp.where(upd, e, i1)
        m2 = jnp.full((16,), -1e30, jnp.float32)
        i2 = jnp.zeros((16,), jnp.int32)
        for e in range(E):
            upd = (ls[e] > m2) & (i1 != e)
            m2 = jnp.where(upd, ls[e], m2)
            i2 = jnp.where(upd, e, i2)
        w1 = 1.0 / (1.0 + jnp.exp(m2 - m1))
        eix_v[pl.ds(base, 16)] = i1
        eix_v[pl.ds(N + base, 16)] = i2
        wts_v[pl.ds(base, 16)] = w1
        wts_v[pl.ds(N + base, 16)] = 1.0 - w1
        return carry

    lax.fori_loop(0, N // 16, route_chunk, 0, unroll=False)

    def count_chunk(c, cnt):
        v = eix_v[pl.ds(c * 16, 16)]
        for e in range(E):
            pc = jnp.sum(jnp.where(v == e, 1, 0))
            cnt = cnt + jnp.where(iota16 == e, pc, 0)
        return cnt

    cnt = lax.fori_loop(0, 2 * N // 16, count_chunk,
                        jnp.zeros((16,), jnp.int32), unroll=False)

    nblk = (cnt + (BT - 1)) // BT
    csum = plsc.cumsum(nblk)
    startblk = csum - nblk
    segstart = startblk * BT

    for j in range(2):
        bid = iota16 + 16 * j
        be = jnp.zeros((16,), jnp.int32)
        for e in range(E):
            sb = jnp.sum(jnp.where(iota16 == e, startblk, 0))
            eb = jnp.sum(jnp.where(iota16 == e, csum, 0))
            be = jnp.where((bid >= sb) & (bid < eb), e, be)
        be_v[pl.ds(16 * j, 16)] = be

    @pl.when(wid == 0)
    def _():
        pltpu.sync_copy(be_v, be_hbm)

    lo = wid * SPT
    for j in range(SPT // 16):
        own_rows[pl.ds(16 * j, 16)] = jnp.zeros((16,), jnp.int32)

    def pos_chunk(c, base_v):
        v = eix_v[pl.ds(c * 16, 16)]
        pos = jnp.zeros((16,), jnp.int32)
        for e in range(E):
            m = v == e
            inc = jnp.where(m, 1, 0)
            s = plsc.cumsum(inc)
            base_e = jnp.sum(jnp.where(iota16 == e, base_v, 0))
            pos = jnp.where(m, base_e + s - 1, pos)
            base_v = base_v + jnp.where(iota16 == e, jnp.sum(inc), 0)
        pos_v[pl.ds(c * 16, 16)] = pos
        c16 = c * 16
        tok = (c16 - (c16 // N) * N) + iota16
        mo = (pos >= lo) & (pos < lo + SPT)
        plsc.store_scatter(own_rows, [pos - lo], tok, mask=mo)
        return base_v

    lax.fori_loop(0, 2 * N // 16, pos_chunk, segstart, unroll=False)

    pltpu.sync_copy(pos_v.at[pl.ds(wid * PPT, PPT)],
                    so_hbm.at[pl.ds(wid * PPT, PPT)])
    pltpu.sync_copy(wts_v.at[pl.ds(wid * PPT, PPT)],
                    tw_hbm.at[pl.ds(wid * PPT, PPT)])

    for j in range(SPT // GB):
        idx = own_rows.at[pl.ds(j * GB, GB)]
        pltpu.async_copy(x_hbm.at[idx], rows_buf, sem).wait()
        pltpu.sync_copy(rows_buf, xs_hbm.at[pl.ds(lo + j * GB, GB)])


def _route_and_gather(lt_flat, x):
    mesh = plsc.VectorSubcoreMesh(**_SC_MESH)
    fn = functools.partial(
        pl.kernel,
        out_type=[
            jax.ShapeDtypeStruct((S, D), jnp.float32),
            jax.ShapeDtypeStruct((NW,), jnp.int32),
            jax.ShapeDtypeStruct((2 * N,), jnp.int32),
            jax.ShapeDtypeStruct((2 * N,), jnp.float32),
        ],
        mesh=mesh,
        compiler_params=pltpu.CompilerParams(needs_layout_passes=False),
        scratch_types=[
            pltpu.VMEM((E * N,), jnp.float32),
            pltpu.VMEM((2 * N,), jnp.int32),
            pltpu.VMEM((2 * N,), jnp.float32),
            pltpu.VMEM((2 * N,), jnp.int32),
            pltpu.VMEM((NW,), jnp.int32),
            pltpu.VMEM((SPT,), jnp.int32),
            pltpu.VMEM((GB, D), jnp.float32),
            pltpu.SemaphoreType.DMA,
        ],
    )(_route_body)
    return fn(lt_flat, x)


def _bdot_t(a, b):
    return jax.lax.dot_general(a, b, (((1,), (1,)), ((), ())),
                               preferred_element_type=jnp.float32)


def _grouped_kernel(be_ref, xs_ref, w1_ref, w3_ref, w2_ref,
                    a1_ref, b1_ref, a3_ref, b3_ref, a2_ref, b2_ref,
                    out_ref):
    xb = xs_ref[...].astype(jnp.bfloat16)
    u1 = _bdot_t(xb, a1_ref[0]).astype(jnp.bfloat16)
    w1 = _bdot_t(xb, w1_ref[...]) + _bdot_t(u1, b1_ref[0])
    u3 = _bdot_t(xb, a3_ref[0]).astype(jnp.bfloat16)
    w3 = _bdot_t(xb, w3_ref[...]) + _bdot_t(u3, b3_ref[0])
    h = (w1 * jax.nn.sigmoid(w1) * w3).astype(jnp.bfloat16)
    u2 = _bdot_t(h, a2_ref[0]).astype(jnp.bfloat16)
    out_ref[...] = _bdot_t(h, w2_ref[...]) + _bdot_t(u2, b2_ref[0])


def _grouped_ffn(be, xs, W1b, W3b, W2b, A1b, B1b, A3b, B3b, A2b, B2b):
    grid_spec = pltpu.PrefetchScalarGridSpec(
        num_scalar_prefetch=1,
        grid=(NB,),
        in_specs=[
            pl.BlockSpec((BT, D), lambda b, be: (b, 0)),
            pl.BlockSpec((DFF, D), lambda b, be: (0, 0)),
            pl.BlockSpec((DFF, D), lambda b, be: (0, 0)),
            pl.BlockSpec((D, DFF), lambda b, be: (0, 0)),
            pl.BlockSpec((1, R, D), lambda b, be: (be[b], 0, 0)),
            pl.BlockSpec((1, DFF, R), lambda b, be: (be[b], 0, 0)),
            pl.BlockSpec((1, R, D), lambda b, be: (be[b], 0, 0)),
            pl.BlockSpec((1, DFF, R), lambda b, be: (be[b], 0, 0)),
            pl.BlockSpec((1, R, DFF), lambda b, be: (be[b], 0, 0)),
            pl.BlockSpec((1, D, R), lambda b, be: (be[b], 0, 0)),
        ],
        out_specs=pl.BlockSpec((BT, D), lambda b, be: (b, 0)),
    )
    return pl.pallas_call(
        _grouped_kernel,
        grid_spec=grid_spec,
        out_shape=jax.ShapeDtypeStruct((S, D), jnp.float32),
        compiler_params=pltpu.CompilerParams(
            dimension_semantics=("arbitrary",),
        ),
    )(be, xs, W1b, W3b, W2b, A1b, B1b, A3b, B3b, A2b, B2b)


def _combine_body(os_hbm, so_hbm, tw_hbm, fin_hbm,
                  so_v, tw_v, r0_v, r1_v, sem):
    cid = lax.axis_index("c")
    sid = lax.axis_index("s")
    wid = sid * 2 + cid
    t0 = wid * TPT

    pltpu.sync_copy(so_hbm.at[pl.ds(t0, TPT)], so_v.at[pl.ds(0, TPT)])
    pltpu.sync_copy(so_hbm.at[pl.ds(N + t0, TPT)], so_v.at[pl.ds(TPT, TPT)])
    pltpu.sync_copy(tw_hbm.at[pl.ds(t0, TPT)], tw_v.at[pl.ds(0, TPT)])
    pltpu.sync_copy(tw_hbm.at[pl.ds(N + t0, TPT)], tw_v.at[pl.ds(TPT, TPT)])

    pltpu.async_copy(os_hbm.at[so_v.at[pl.ds(0, TPT)]], r0_v, sem).wait()
    pltpu.async_copy(os_hbm.at[so_v.at[pl.ds(TPT, TPT)]], r1_v, sem).wait()

    def tok_loop(i, carry):
        w0 = tw_v[pl.ds(i, 16)][0]
        w1 = tw_v[pl.ds(TPT + i, 16)][0]
        for j in range(D // 16):
            sl = pl.ds(j * 16, 16)
            r0_v[i, sl] = r0_v[i, sl] * w0 + r1_v[i, sl] * w1
        return carry

    lax.fori_loop(0, TPT, tok_loop, 0, unroll=False)
    pltpu.sync_copy(r0_v, fin_hbm.at[pl.ds(t0, TPT)])


def _combine(os, so, tw):
    mesh = plsc.VectorSubcoreMesh(**_SC_MESH)
    fn = functools.partial(
        pl.kernel,
        out_type=jax.ShapeDtypeStruct((N, D), jnp.float32),
        mesh=mesh,
        compiler_params=pltpu.CompilerParams(needs_layout_passes=False),
        scratch_types=[
            pltpu.VMEM((2 * TPT,), jnp.int32),
            pltpu.VMEM((2 * TPT + 16,), jnp.float32),
            pltpu.VMEM((TPT, D), jnp.float32),
            pltpu.VMEM((TPT, D), jnp.float32),
            pltpu.SemaphoreType.DMA,
        ],
    )(_combine_body)
    return fn(os, so, tw)


@jax.jit
def kernel(score_norm_data, W1, W3, W2, Wg, A1, B1, A3, B3, A2, B2):
    x = score_norm_data
    bf = jnp.bfloat16
    lt = _router_logits(x, Wg).reshape(E * N)
    xs, be, so, tw = _route_and_gather(lt, x)
    os = _grouped_ffn(be, xs,
                      W1.astype(bf), W3.astype(bf), W2.astype(bf),
                      A1.astype(bf), B1.astype(bf), A3.astype(bf),
                      B3.astype(bf), A2.astype(bf), B2.astype(bf))
    return _combine(os, so, tw)
